# Initial kernel scaffold; baseline (speedup 1.0000x reference)
import functools

import jax
import jax.numpy as jnp
from jax import lax
from jax.experimental import pallas as pl
from jax.experimental.pallas import tpu as pltpu
from jax.experimental.pallas import tpu_sc as plsc

N = 50000
D = 64
HALF = 32
E = 800000
NC = 2
NS = 16
LANES = 16

ROW_W = 80
ROWS_PER_TILE = E // NS // ROW_W
CHUNK_ROWS = 25
N_CHUNKS = ROWS_PER_TILE // CHUNK_ROWS
CHUNK_EDGES = CHUNK_ROWS * ROW_W
ACC_ROWS_PER_TILE = N // NS
ZROWS = 625


def _sc_body(dst_hbm, src_hbm, w_hbm, h0_hbm, h1_hbm, h2_hbm,
             idxb, dstb, wb, rows, zbuf, acc, sem):
    c = lax.axis_index("c")
    s = lax.axis_index("s")
    coff = c * N
    wbase = s * ACC_ROWS_PER_TILE

    @pl.loop(0, ZROWS, unroll=1)
    def _(r):
        for k in range(HALF // LANES):
            zbuf[r, pl.ds(k * LANES, LANES)] = jnp.zeros((LANES,), jnp.float32)

    def zero_acc():
        for j in range(ACC_ROWS_PER_TILE // ZROWS):
            pltpu.sync_copy(zbuf, acc.at[pl.ds(wbase + j * ZROWS, ZROWS)])

    def run_layer(h_in):
        @pl.loop(0, N_CHUNKS, unroll=1)
        def _(chunk):
            base_row = s * ROWS_PER_TILE + chunk * CHUNK_ROWS
            pltpu.sync_copy(src_hbm.at[pl.ds(base_row, CHUNK_ROWS)], idxb)
            pltpu.sync_copy(dst_hbm.at[pl.ds(base_row, CHUNK_ROWS)], dstb)
            pltpu.sync_copy(w_hbm.at[pl.ds(base_row, CHUNK_ROWS)], wb)

            @pl.loop(0, CHUNK_ROWS, unroll=1)
            def _(r):
                for k in range(ROW_W // LANES):
                    sl = pl.ds(k * LANES, LANES)
                    idxb[r, sl] = idxb[r, sl] + coff

            descs = [
                pltpu.async_copy(h_in.at[idxb.at[i]],
                                 rows.at[pl.ds(i * ROW_W, ROW_W)], sem)
                for i in range(CHUNK_ROWS)
            ]
            for d in descs:
                d.wait()

            @pl.loop(0, CHUNK_ROWS, unroll=1)
            def _(r):
                ebase = r * ROW_W
                for k in range(ROW_W // LANES):
                    w16 = wb[r, pl.ds(k * LANES, LANES)]
                    ev = lax.iota(jnp.int32, (LANES,)) + (ebase + k * LANES)
                    for j in range(HALF):
                        cj = jnp.full((LANES,), j, jnp.int32)
                        v = plsc.load_gather(rows, [ev, cj])
                        plsc.store_scatter(rows, [ev, cj], v * w16)

            @pl.loop(0, CHUNK_ROWS, unroll=1)
            def _(i):
                pltpu.sync_copy(rows.at[pl.ds(i * ROW_W, ROW_W)],
                                acc.at[dstb.at[i]], add=True)

    zero_acc()
    plsc.subcore_barrier()
    run_layer(h0_hbm)
    plsc.subcore_barrier()
    pltpu.sync_copy(acc.at[pl.ds(wbase, ACC_ROWS_PER_TILE)],
                    h1_hbm.at[pl.ds(coff + wbase, ACC_ROWS_PER_TILE)])
    zero_acc()
    plsc.subcore_barrier()
    run_layer(h1_hbm)
    plsc.subcore_barrier()
    pltpu.sync_copy(acc.at[pl.ds(wbase, ACC_ROWS_PER_TILE)],
                    h2_hbm.at[pl.ds(coff + wbase, ACC_ROWS_PER_TILE)])


_propagate = functools.partial(
    pl.kernel,
    out_type=(jax.ShapeDtypeStruct((2 * N, HALF), jnp.float32),
              jax.ShapeDtypeStruct((2 * N, HALF), jnp.float32)),
    mesh=plsc.VectorSubcoreMesh(core_axis_name="c", subcore_axis_name="s"),
    scratch_types=[
        pltpu.VMEM((CHUNK_ROWS, ROW_W), jnp.int32),
        pltpu.VMEM((CHUNK_ROWS, ROW_W), jnp.int32),
        pltpu.VMEM((CHUNK_ROWS, ROW_W), jnp.float32),
        pltpu.VMEM((CHUNK_EDGES, HALF), jnp.float32),
        pltpu.VMEM((ZROWS, HALF), jnp.float32),
        pltpu.VMEM_SHARED((N, HALF), jnp.float32),
        pltpu.SemaphoreType.DMA,
    ],
)(_sc_body)


_MEAN_BLK = 2000
_MEAN_GRID = N // _MEAN_BLK


def _mean_body(e_ref, a_lo, a_hi, b_lo, b_hi, o_ref):
    third = jnp.float32(1.0 / 3.0)
    o_ref[:, 0:HALF] = (e_ref[:, 0:HALF] + a_lo[...] + b_lo[...]) * third
    o_ref[:, HALF:D] = (e_ref[:, HALF:D] + a_hi[...] + b_hi[...]) * third


def _mean3(emb, h1, h2):
    half_spec_lo = pl.BlockSpec((_MEAN_BLK, HALF), lambda i: (i, 0))
    half_spec_hi = pl.BlockSpec((_MEAN_BLK, HALF), lambda i: (i + _MEAN_GRID, 0))
    return pl.pallas_call(
        _mean_body,
        out_shape=jax.ShapeDtypeStruct((N, D), jnp.float32),
        grid=(_MEAN_GRID,),
        in_specs=[
            pl.BlockSpec((_MEAN_BLK, D), lambda i: (i, 0)),
            half_spec_lo, half_spec_hi,
            half_spec_lo, half_spec_hi,
        ],
        out_specs=pl.BlockSpec((_MEAN_BLK, D), lambda i: (i, 0)),
    )(emb, h1, h1, h2, h2)


def kernel(edge_index, edge_weight, emb_weight):
    dst = edge_index[0].reshape(E // ROW_W, ROW_W)
    src = edge_index[1].reshape(E // ROW_W, ROW_W)
    w = edge_weight.reshape(E // ROW_W, ROW_W)
    h0 = jnp.concatenate([emb_weight[:, :HALF], emb_weight[:, HALF:]], axis=0)
    h1, h2 = _propagate(dst, src, w, h0)
    return _mean3(emb_weight, h1, h2)


# speedup vs baseline: 5.8336x; 5.8336x over previous
import functools

import jax
import jax.numpy as jnp
from jax import lax
from jax.experimental import pallas as pl
from jax.experimental.pallas import tpu as pltpu
from jax.experimental.pallas import tpu_sc as plsc

N = 50000
N_PAD = 50048
D = 64
HALF = 32
E = 800000
NS = 16
LANES = 16

ROW_W = 80
E_PER_TILE = E // NS
CHUNK_EDGES = 400
CHUNK_ROWS = CHUNK_EDGES // ROW_W
N_CHUNKS = E_PER_TILE // CHUNK_EDGES
ACC_ROWS_PER_TILE = N_PAD // NS
ZROWS = 184


def _sc_body(dst_hbm, src_hbm, w_hbm, h0_hbm, h1_hbm, h2_hbm,
             srcb, dstb1, dstb2, wb, rows, zbuf, acc, sem):
    c = lax.axis_index("c")
    s = lax.axis_index("s")
    coff = c * N_PAD
    wbase = s * ACC_ROWS_PER_TILE

    @pl.loop(0, ZROWS, unroll=1)
    def _(r):
        for k in range(HALF // LANES):
            zbuf[r, pl.ds(k * LANES, LANES)] = jnp.zeros((LANES,), jnp.float32)

    def zero_acc():
        for j in range(ACC_ROWS_PER_TILE // ZROWS):
            pltpu.sync_copy(zbuf, acc.at[pl.ds(wbase + j * ZROWS, ZROWS)])

    def run_layer(h_in):
        @pl.loop(0, N_CHUNKS, unroll=1)
        def _(chunk):
            ebase_hbm = s * E_PER_TILE + chunk * CHUNK_EDGES
            pltpu.sync_copy(src_hbm.at[pl.ds(ebase_hbm, CHUNK_EDGES)], srcb)
            pltpu.sync_copy(dst_hbm.at[pl.ds(ebase_hbm, CHUNK_EDGES)], dstb1)
            pltpu.sync_copy(w_hbm.at[pl.ds(ebase_hbm, CHUNK_EDGES)], wb)

            @pl.loop(0, CHUNK_ROWS, unroll=1)
            def _(r):
                for k in range(ROW_W // LANES):
                    p = pl.ds(r * ROW_W + k * LANES, LANES)
                    srcb[p] = srcb[p] + coff
                    dstb2[r, pl.ds(k * LANES, LANES)] = dstb1[p]

            descs = [
                pltpu.async_copy(h_in.at[srcb.at[pl.ds(i * ROW_W, ROW_W)]],
                                 rows.at[pl.ds(i * ROW_W, ROW_W)], sem)
                for i in range(CHUNK_ROWS)
            ]
            for d in descs:
                d.wait()

            @pl.loop(0, CHUNK_EDGES // LANES, unroll=1)
            def _(g):
                w16 = wb[pl.ds(g * LANES, LANES)]
                for l in range(LANES):
                    e = g * LANES + l
                    wsc = w16[l]
                    for k in range(HALF // LANES):
                        sl = pl.ds(k * LANES, LANES)
                        rows[e, sl] = rows[e, sl] * wsc

            @pl.loop(0, CHUNK_ROWS, unroll=1)
            def _(i):
                pltpu.sync_copy(rows.at[pl.ds(i * ROW_W, ROW_W)],
                                acc.at[dstb2.at[i]], add=True)

    zero_acc()
    plsc.subcore_barrier()
    run_layer(h0_hbm)
    plsc.subcore_barrier()
    pltpu.sync_copy(acc.at[pl.ds(wbase, ACC_ROWS_PER_TILE)],
                    h1_hbm.at[pl.ds(coff + wbase, ACC_ROWS_PER_TILE)])
    zero_acc()
    plsc.subcore_barrier()
    run_layer(h1_hbm)
    plsc.subcore_barrier()
    pltpu.sync_copy(acc.at[pl.ds(wbase, ACC_ROWS_PER_TILE)],
                    h2_hbm.at[pl.ds(coff + wbase, ACC_ROWS_PER_TILE)])


_propagate = functools.partial(
    pl.kernel,
    out_type=(jax.ShapeDtypeStruct((2 * N_PAD, HALF), jnp.float32),
              jax.ShapeDtypeStruct((2 * N_PAD, HALF), jnp.float32)),
    mesh=plsc.VectorSubcoreMesh(core_axis_name="c", subcore_axis_name="s"),
    compiler_params=pltpu.CompilerParams(use_tc_tiling_on_sc=False),
    scratch_types=[
        pltpu.VMEM((CHUNK_EDGES,), jnp.int32),
        pltpu.VMEM((CHUNK_EDGES,), jnp.int32),
        pltpu.VMEM((CHUNK_ROWS, ROW_W), jnp.int32),
        pltpu.VMEM((CHUNK_EDGES,), jnp.float32),
        pltpu.VMEM((CHUNK_EDGES, HALF), jnp.float32),
        pltpu.VMEM((ZROWS, HALF), jnp.float32),
        pltpu.VMEM_SHARED((N_PAD, HALF), jnp.float32),
        pltpu.SemaphoreType.DMA,
    ],
)(_sc_body)


_MEAN_BLK = 2000
_MEAN_GRID = N // _MEAN_BLK


def _mean_body(e_ref, a_lo, a_hi, b_lo, b_hi, o_ref):
    third = jnp.float32(1.0 / 3.0)
    o_ref[:, 0:HALF] = (e_ref[:, 0:HALF] + a_lo[...] + b_lo[...]) * third
    o_ref[:, HALF:D] = (e_ref[:, HALF:D] + a_hi[...] + b_hi[...]) * third


def _mean3(emb, a_lo, a_hi, b_lo, b_hi):
    half_spec = pl.BlockSpec((_MEAN_BLK, HALF), lambda i: (i, 0))
    return pl.pallas_call(
        _mean_body,
        out_shape=jax.ShapeDtypeStruct((N, D), jnp.float32),
        grid=(_MEAN_GRID,),
        in_specs=[
            pl.BlockSpec((_MEAN_BLK, D), lambda i: (i, 0)),
            half_spec, half_spec, half_spec, half_spec,
        ],
        out_specs=pl.BlockSpec((_MEAN_BLK, D), lambda i: (i, 0)),
    )(emb, a_lo, a_hi, b_lo, b_hi)


def kernel(edge_index, edge_weight, emb_weight):
    dst = edge_index[0]
    src = edge_index[1]
    pad = jnp.zeros((N_PAD - N, HALF), jnp.float32)
    h0 = jnp.concatenate(
        [emb_weight[:, :HALF], pad, emb_weight[:, HALF:], pad], axis=0)
    h1, h2 = _propagate(dst, src, edge_weight, h0)
    return _mean3(emb_weight,
                  h1[:N], h1[N_PAD:N_PAD + N],
                  h2[:N], h2[N_PAD:N_PAD + N])


# speedup vs baseline: 8.2734x; 1.4182x over previous
import functools

import jax
import jax.numpy as jnp
from jax import lax
from jax.experimental import pallas as pl
from jax.experimental.pallas import tpu as pltpu
from jax.experimental.pallas import tpu_sc as plsc

N = 50000
N_PAD = 50048
D = 64
HALF = 32
E = 800000
NS = 16
LANES = 16

ROW_W = 80
NDESC = 5
CHUNK_EDGES = NDESC * ROW_W
EROWS = E // ROW_W
ROWS_PER_TILE = EROWS // NS
N_CHUNKS = ROWS_PER_TILE // NDESC
N_PAIRS = (N_CHUNKS - 1) // 2
ACC_ROWS_PER_TILE = N_PAD // NS


def _sc_body(dst_hbm, src_hbm, w_hbm, h0_hbm, h1_hbm, h2_hbm,
             srcb0, srcb1, dstb0, dstb1, wb0, wb1, rows0, rows1, acc,
             esem0, esem1, gsem0, gsem1, ssem0, ssem1):
    c = lax.axis_index("c")
    s = lax.axis_index("s")
    coff = c * N_PAD
    wbase = s * ACC_ROWS_PER_TILE
    srcb = (srcb0, srcb1)
    dstb = (dstb0, dstb1)
    wb = (wb0, wb1)
    rows = (rows0, rows1)
    esem = (esem0, esem1)
    gsem = (gsem0, gsem1)
    ssem = (ssem0, ssem1)

    def make_pipeline(acc):
        def load_edges(pi, chunk):
            base = s * ROWS_PER_TILE + chunk * NDESC
            pltpu.async_copy(src_hbm.at[pl.ds(base, NDESC)], srcb[pi], esem[pi])
            pltpu.async_copy(dst_hbm.at[pl.ds(base, NDESC)], dstb[pi], esem[pi])
            pltpu.async_copy(w_hbm.at[pl.ds(base, NDESC)], wb[pi], esem[pi])

        def wait_edges(pi):
            pltpu.make_async_copy(src_hbm.at[pl.ds(0, NDESC)], srcb[pi], esem[pi]).wait()
            pltpu.make_async_copy(dst_hbm.at[pl.ds(0, NDESC)], dstb[pi], esem[pi]).wait()
            pltpu.make_async_copy(w_hbm.at[pl.ds(0, NDESC)], wb[pi], esem[pi]).wait()

        def prep_idx(pi):
            @pl.loop(0, NDESC, unroll=1)
            def _(r):
                for k in range(ROW_W // LANES):
                    sl = pl.ds(k * LANES, LANES)
                    srcb[pi][r, sl] = srcb[pi][r, sl] + coff

        def fire_gathers(pi, h_in):
            for i in range(NDESC):
                pltpu.async_copy(h_in.at[srcb[pi].at[i]],
                                 rows[pi].at[pl.ds(i * ROW_W, ROW_W)], gsem[pi])

        def wait_gathers(pi, h_in):
            for i in range(NDESC):
                pltpu.make_async_copy(h_in.at[srcb[pi].at[i]],
                                      rows[pi].at[pl.ds(i * ROW_W, ROW_W)],
                                      gsem[pi]).wait()

        def multiply(pi):
            @pl.loop(0, NDESC, unroll=1)
            def _(i):
                for g in range(ROW_W // LANES):
                    w16 = wb[pi][i, pl.ds(g * LANES, LANES)]
                    for l in range(LANES):
                        e = i * ROW_W + g * LANES + l
                        wsc = w16[l]
                        for k in range(HALF // LANES):
                            sl = pl.ds(k * LANES, LANES)
                            rows[pi][e, sl] = rows[pi][e, sl] * wsc

        def fire_scatters(pi):
            for i in range(NDESC):
                pltpu.async_copy(rows[pi].at[pl.ds(i * ROW_W, ROW_W)],
                                 acc.at[dstb[pi].at[i]], ssem[pi], add=True)

        def wait_scatters(pi):
            for i in range(NDESC):
                pltpu.make_async_copy(rows[pi].at[pl.ds(i * ROW_W, ROW_W)],
                                      acc.at[dstb[pi].at[i]], ssem[pi]).wait()

        def run_layer(h_in):
            load_edges(0, 0)
            wait_edges(0)
            prep_idx(0)
            fire_gathers(0, h_in)

            @pl.loop(0, N_PAIRS, unroll=1)
            def _(t):
                c0 = 2 * t
                wait_gathers(0, h_in)
                multiply(0)
                fire_scatters(0)

                @pl.when(t > 0)
                def _():
                    wait_scatters(1)
                load_edges(1, c0 + 1)
                wait_edges(1)
                prep_idx(1)
                fire_gathers(1, h_in)

                wait_gathers(1, h_in)
                multiply(1)
                fire_scatters(1)

                wait_scatters(0)
                load_edges(0, c0 + 2)
                wait_edges(0)
                prep_idx(0)
                fire_gathers(0, h_in)

            wait_gathers(0, h_in)
            multiply(0)
            fire_scatters(0)
            wait_scatters(1)
            wait_scatters(0)

        def zero_rows0():
            @pl.loop(0, CHUNK_EDGES, unroll=4)
            def _(r):
                for k in range(HALF // LANES):
                    rows0[r, pl.ds(k * LANES, LANES)] = jnp.zeros((LANES,), jnp.float32)

        def zero_acc():
            for j in range(ACC_ROWS_PER_TILE // CHUNK_EDGES):
                pltpu.sync_copy(rows0, acc.at[pl.ds(wbase + j * CHUNK_EDGES, CHUNK_EDGES)])
            rem = ACC_ROWS_PER_TILE % CHUNK_EDGES
            pltpu.sync_copy(
                rows0.at[pl.ds(0, rem)],
                acc.at[pl.ds(wbase + ACC_ROWS_PER_TILE - rem, rem)])

        def writeout(h_out):
            pltpu.sync_copy(acc.at[pl.ds(wbase, ACC_ROWS_PER_TILE)],
                            h_out.at[pl.ds(coff + wbase, ACC_ROWS_PER_TILE)])

        return run_layer, zero_rows0, zero_acc, writeout

    run_layer, zero_rows0, zero_acc, writeout = make_pipeline(acc)
    zero_rows0()
    zero_acc()
    plsc.subcore_barrier()
    run_layer(h0_hbm)
    plsc.subcore_barrier()
    writeout(h1_hbm)
    zero_rows0()
    zero_acc()
    plsc.subcore_barrier()
    run_layer(h1_hbm)
    plsc.subcore_barrier()
    writeout(h2_hbm)


_propagate = functools.partial(
    pl.kernel,
    out_type=(jax.ShapeDtypeStruct((2 * N_PAD, HALF), jnp.float32),
              jax.ShapeDtypeStruct((2 * N_PAD, HALF), jnp.float32)),
    mesh=plsc.VectorSubcoreMesh(core_axis_name="c", subcore_axis_name="s"),
    compiler_params=pltpu.CompilerParams(use_tc_tiling_on_sc=False),
    scratch_types=[
        pltpu.VMEM((NDESC, ROW_W), jnp.int32),
        pltpu.VMEM((NDESC, ROW_W), jnp.int32),
        pltpu.VMEM((NDESC, ROW_W), jnp.int32),
        pltpu.VMEM((NDESC, ROW_W), jnp.int32),
        pltpu.VMEM((NDESC, ROW_W), jnp.float32),
        pltpu.VMEM((NDESC, ROW_W), jnp.float32),
        pltpu.VMEM((CHUNK_EDGES, HALF), jnp.float32),
        pltpu.VMEM((CHUNK_EDGES, HALF), jnp.float32),
        pltpu.VMEM_SHARED((N_PAD, HALF), jnp.float32),
        pltpu.SemaphoreType.DMA,
        pltpu.SemaphoreType.DMA,
        pltpu.SemaphoreType.DMA,
        pltpu.SemaphoreType.DMA,
        pltpu.SemaphoreType.DMA,
        pltpu.SemaphoreType.DMA,
    ],
)(_sc_body)


_MEAN_BLK = 2000
_MEAN_GRID = N // _MEAN_BLK


def _mean_body(e_ref, a_lo, a_hi, b_lo, b_hi, o_ref):
    third = jnp.float32(1.0 / 3.0)
    o_ref[:, 0:HALF] = (e_ref[:, 0:HALF] + a_lo[...] + b_lo[...]) * third
    o_ref[:, HALF:D] = (e_ref[:, HALF:D] + a_hi[...] + b_hi[...]) * third


def _mean3(emb, a_lo, a_hi, b_lo, b_hi):
    half_spec = pl.BlockSpec((_MEAN_BLK, HALF), lambda i: (i, 0))
    return pl.pallas_call(
        _mean_body,
        out_shape=jax.ShapeDtypeStruct((N, D), jnp.float32),
        grid=(_MEAN_GRID,),
        in_specs=[
            pl.BlockSpec((_MEAN_BLK, D), lambda i: (i, 0)),
            half_spec, half_spec, half_spec, half_spec,
        ],
        out_specs=pl.BlockSpec((_MEAN_BLK, D), lambda i: (i, 0)),
    )(emb, a_lo, a_hi, b_lo, b_hi)


def kernel(edge_index, edge_weight, emb_weight):
    dst = edge_index[0].reshape(EROWS, ROW_W)
    src = edge_index[1].reshape(EROWS, ROW_W)
    w = edge_weight.reshape(EROWS, ROW_W)
    pad = jnp.zeros((N_PAD - N, HALF), jnp.float32)
    h0 = jnp.concatenate(
        [emb_weight[:, :HALF], pad, emb_weight[:, HALF:], pad], axis=0)
    h1, h2 = _propagate(dst, src, w, h0)
    return _mean3(emb_weight,
                  h1[:N], h1[N_PAD:N_PAD + N],
                  h2[:N], h2[N_PAD:N_PAD + N])


# speedup vs baseline: 12.3578x; 1.4937x over previous
import functools

import jax
import jax.numpy as jnp
from jax import lax
from jax.experimental import pallas as pl
from jax.experimental.pallas import tpu as pltpu
from jax.experimental.pallas import tpu_sc as plsc

N = 50000
N_PAD = 50048
D = 64
HALF = 32
E = 800000
NS = 16
LANES = 16

ROW_W = 80
NDESC = 5
CHUNK_EDGES = NDESC * ROW_W
EROWS = E // ROW_W
ROWS_PER_TILE = EROWS // NS
N_CHUNKS = ROWS_PER_TILE // NDESC
ESLOTS = 4
ACC_ROWS_PER_TILE = N_PAD // NS


def _sc_body(dst_hbm, src_hbm, w_hbm, h0_hbm, h1_hbm, h2_hbm,
             srcb, dstb, wbuf, rows, acc, esem, gsem, ssem):
    c = lax.axis_index("c")
    s = lax.axis_index("s")
    coff = c * N_PAD
    wbase = s * ACC_ROWS_PER_TILE

    def load_edges(slot, chunk):
        base = s * ROWS_PER_TILE + chunk * NDESC
        row = slot * NDESC
        pltpu.async_copy(src_hbm.at[pl.ds(base, NDESC)],
                         srcb.at[pl.ds(row, NDESC)], esem)
        pltpu.async_copy(dst_hbm.at[pl.ds(base, NDESC)],
                         dstb.at[pl.ds(row, NDESC)], esem)
        pltpu.async_copy(w_hbm.at[pl.ds(base, NDESC)],
                         wbuf.at[pl.ds(row, NDESC)], esem)

    def wait_edges():
        pltpu.make_async_copy(src_hbm.at[pl.ds(0, NDESC)],
                              srcb.at[pl.ds(0, NDESC)], esem).wait()
        pltpu.make_async_copy(dst_hbm.at[pl.ds(0, NDESC)],
                              dstb.at[pl.ds(0, NDESC)], esem).wait()
        pltpu.make_async_copy(w_hbm.at[pl.ds(0, NDESC)],
                              wbuf.at[pl.ds(0, NDESC)], esem).wait()

    def prep_idx(slot):
        row = slot * NDESC
        for r in range(NDESC):
            for k in range(ROW_W // LANES):
                sl = pl.ds(k * LANES, LANES)
                srcb[row + r, sl] = srcb[row + r, sl] + coff

    def fire_gathers(h_in, slot, rbase):
        for i in range(NDESC):
            pltpu.async_copy(h_in.at[srcb.at[slot * NDESC + i]],
                             rows.at[pl.ds(rbase + i * ROW_W, ROW_W)], gsem)

    def wait_gathers(h_in):
        for i in range(NDESC):
            pltpu.make_async_copy(h_in.at[srcb.at[i]],
                                  rows.at[pl.ds(i * ROW_W, ROW_W)], gsem).wait()

    def fire_scatters(slot, rbase):
        for i in range(NDESC):
            pltpu.async_copy(rows.at[pl.ds(rbase + i * ROW_W, ROW_W)],
                             acc.at[dstb.at[slot * NDESC + i]], ssem, add=True)

    def wait_scatters():
        for i in range(NDESC):
            pltpu.make_async_copy(rows.at[pl.ds(i * ROW_W, ROW_W)],
                                  acc.at[dstb.at[i]], ssem).wait()

    def run_layer(h_in):
        load_edges(0, 0)
        wait_edges()
        prep_idx(0)
        fire_gathers(h_in, 0, 0)
        load_edges(1, 1)

        @pl.loop(0, N_CHUNKS, unroll=1)
        def _(k):
            par = k & 1
            rbase = par * CHUNK_EDGES
            rbase_n = (1 - par) * CHUNK_EDGES
            slot = k & 3
            slot_n = (k + 1) & 3
            last = N_CHUNKS - 1

            @pl.when(k < last)
            def _():
                wait_edges()
                prep_idx(slot_n)

            @pl.when(k < N_CHUNKS - 2)
            def _():
                load_edges((k + 2) & 3, k + 2)

            @pl.when(k > 0)
            def _():
                wait_scatters()

            @pl.when(k < last)
            def _():
                fire_gathers(h_in, slot_n, rbase_n)

            wait_gathers(h_in)

            for i in range(NDESC):
                for g in range(ROW_W // LANES):
                    w16 = wbuf[slot * NDESC + i, pl.ds(g * LANES, LANES)]
                    for l in range(LANES):
                        e = rbase + i * ROW_W + g * LANES + l
                        wsc = w16[l]
                        for kk in range(HALF // LANES):
                            sl = pl.ds(kk * LANES, LANES)
                            rows[e, sl] = rows[e, sl] * wsc
                pltpu.async_copy(rows.at[pl.ds(rbase + i * ROW_W, ROW_W)],
                                 acc.at[dstb.at[slot * NDESC + i]], ssem,
                                 add=True)

        wait_scatters()

    def zero_rows():
        @pl.loop(0, 2 * CHUNK_EDGES, unroll=4)
        def _(r):
            for k in range(HALF // LANES):
                rows[r, pl.ds(k * LANES, LANES)] = jnp.zeros((LANES,), jnp.float32)

    def zero_acc():
        nz = 2 * CHUNK_EDGES
        for j in range(ACC_ROWS_PER_TILE // nz):
            pltpu.sync_copy(rows, acc.at[pl.ds(wbase + j * nz, nz)])
        rem = ACC_ROWS_PER_TILE % nz
        pltpu.sync_copy(rows.at[pl.ds(0, rem)],
                        acc.at[pl.ds(wbase + ACC_ROWS_PER_TILE - rem, rem)])

    def writeout(h_out):
        pltpu.sync_copy(acc.at[pl.ds(wbase, ACC_ROWS_PER_TILE)],
                        h_out.at[pl.ds(coff + wbase, ACC_ROWS_PER_TILE)])

    zero_rows()
    zero_acc()
    plsc.subcore_barrier()
    run_layer(h0_hbm)
    plsc.subcore_barrier()
    writeout(h1_hbm)
    zero_rows()
    zero_acc()
    plsc.subcore_barrier()
    run_layer(h1_hbm)
    plsc.subcore_barrier()
    writeout(h2_hbm)


_propagate = functools.partial(
    pl.kernel,
    out_type=(jax.ShapeDtypeStruct((2 * N_PAD, HALF), jnp.float32),
              jax.ShapeDtypeStruct((2 * N_PAD, HALF), jnp.float32)),
    mesh=plsc.VectorSubcoreMesh(core_axis_name="c", subcore_axis_name="s"),
    compiler_params=pltpu.CompilerParams(use_tc_tiling_on_sc=False),
    scratch_types=[
        pltpu.VMEM((ESLOTS * NDESC, ROW_W), jnp.int32),
        pltpu.VMEM((ESLOTS * NDESC, ROW_W), jnp.int32),
        pltpu.VMEM((ESLOTS * NDESC, ROW_W), jnp.float32),
        pltpu.VMEM((2 * CHUNK_EDGES, HALF), jnp.float32),
        pltpu.VMEM_SHARED((N_PAD, HALF), jnp.float32),
        pltpu.SemaphoreType.DMA,
        pltpu.SemaphoreType.DMA,
        pltpu.SemaphoreType.DMA,
    ],
)(_sc_body)


_MEAN_BLK = 2000
_MEAN_GRID = N // _MEAN_BLK


def _mean_body(e_ref, a_lo, a_hi, b_lo, b_hi, o_ref):
    third = jnp.float32(1.0 / 3.0)
    o_ref[:, 0:HALF] = (e_ref[:, 0:HALF] + a_lo[...] + b_lo[...]) * third
    o_ref[:, HALF:D] = (e_ref[:, HALF:D] + a_hi[...] + b_hi[...]) * third


def _mean3(emb, a_lo, a_hi, b_lo, b_hi):
    half_spec = pl.BlockSpec((_MEAN_BLK, HALF), lambda i: (i, 0))
    return pl.pallas_call(
        _mean_body,
        out_shape=jax.ShapeDtypeStruct((N, D), jnp.float32),
        grid=(_MEAN_GRID,),
        in_specs=[
            pl.BlockSpec((_MEAN_BLK, D), lambda i: (i, 0)),
            half_spec, half_spec, half_spec, half_spec,
        ],
        out_specs=pl.BlockSpec((_MEAN_BLK, D), lambda i: (i, 0)),
    )(emb, a_lo, a_hi, b_lo, b_hi)


def kernel(edge_index, edge_weight, emb_weight):
    dst = edge_index[0].reshape(EROWS, ROW_W)
    src = edge_index[1].reshape(EROWS, ROW_W)
    w = edge_weight.reshape(EROWS, ROW_W)
    pad = jnp.zeros((N_PAD - N, HALF), jnp.float32)
    h0 = jnp.concatenate(
        [emb_weight[:, :HALF], pad, emb_weight[:, HALF:], pad], axis=0)
    h1, h2 = _propagate(dst, src, w, h0)
    return _mean3(emb_weight,
                  h1[:N], h1[N_PAD:N_PAD + N],
                  h2[:N], h2[N_PAD:N_PAD + N])


# speedup vs baseline: 17.1719x; 1.3896x over previous
import functools

import jax
import jax.numpy as jnp
from jax import lax
from jax.experimental import pallas as pl
from jax.experimental.pallas import tpu as pltpu
from jax.experimental.pallas import tpu_sc as plsc

N = 50000
N_PAD = 50048
D = 64
HALF = 32
E = 800000
NS = 16
LANES = 16

ROW_W = 80
NDESC = 5
CHUNK_EDGES = NDESC * ROW_W
EROWS = E // ROW_W
ROWS_PER_TILE = EROWS // NS
N_CHUNKS = ROWS_PER_TILE // NDESC
ESLOTS = 4
ACC_ROWS_PER_TILE = N_PAD // NS


def _sc_body(dst_hbm, src_hbm, w_hbm, emb_hbm, h0_hbm, h1_hbm, out_hbm,
             srcb, dstb, wbuf, rows, acc, esem, gsem, ssem):
    c = lax.axis_index("c")
    s = lax.axis_index("s")
    coff = c * N_PAD
    wbase = s * ACC_ROWS_PER_TILE
    N_LAST = N - (NS - 1) * ACC_ROWS_PER_TILE

    def load_edges(slot, chunk):
        base = s * ROWS_PER_TILE + chunk * NDESC
        row = slot * NDESC
        pltpu.async_copy(src_hbm.at[pl.ds(base, NDESC)],
                         srcb.at[pl.ds(row, NDESC)], esem)
        pltpu.async_copy(dst_hbm.at[pl.ds(base, NDESC)],
                         dstb.at[pl.ds(row, NDESC)], esem)
        pltpu.async_copy(w_hbm.at[pl.ds(base, NDESC)],
                         wbuf.at[pl.ds(row, NDESC)], esem)

    def wait_edges():
        pltpu.make_async_copy(src_hbm.at[pl.ds(0, NDESC)],
                              srcb.at[pl.ds(0, NDESC)], esem).wait()
        pltpu.make_async_copy(dst_hbm.at[pl.ds(0, NDESC)],
                              dstb.at[pl.ds(0, NDESC)], esem).wait()
        pltpu.make_async_copy(w_hbm.at[pl.ds(0, NDESC)],
                              wbuf.at[pl.ds(0, NDESC)], esem).wait()

    def prep_idx(slot):
        row = slot * NDESC
        for r in range(NDESC):
            for k in range(ROW_W // LANES):
                sl = pl.ds(k * LANES, LANES)
                srcb[row + r, sl] = srcb[row + r, sl] + coff

    def fire_gathers(h_in, slot, rbase):
        for i in range(NDESC):
            pltpu.async_copy(h_in.at[srcb.at[slot * NDESC + i]],
                             rows.at[pl.ds(rbase + i * ROW_W, ROW_W)], gsem)

    def wait_gathers(h_in):
        for i in range(NDESC):
            pltpu.make_async_copy(h_in.at[srcb.at[i]],
                                  rows.at[pl.ds(i * ROW_W, ROW_W)], gsem).wait()

    def fire_scatters(slot, rbase):
        for i in range(NDESC):
            pltpu.async_copy(rows.at[pl.ds(rbase + i * ROW_W, ROW_W)],
                             acc.at[dstb.at[slot * NDESC + i]], ssem, add=True)

    def wait_scatters():
        for i in range(NDESC):
            pltpu.make_async_copy(rows.at[pl.ds(i * ROW_W, ROW_W)],
                                  acc.at[dstb.at[i]], ssem).wait()

    def run_layer(h_in):
        load_edges(0, 0)
        wait_edges()
        prep_idx(0)
        fire_gathers(h_in, 0, 0)
        load_edges(1, 1)

        @pl.loop(0, N_CHUNKS, unroll=1)
        def _(k):
            par = k & 1
            rbase = par * CHUNK_EDGES
            rbase_n = (1 - par) * CHUNK_EDGES
            slot = k & 3
            slot_n = (k + 1) & 3
            last = N_CHUNKS - 1

            @pl.when(k < last)
            def _():
                wait_edges()
                prep_idx(slot_n)

            @pl.when(k < N_CHUNKS - 2)
            def _():
                load_edges((k + 2) & 3, k + 2)

            @pl.when(k > 0)
            def _():
                wait_scatters()

            @pl.when(k < last)
            def _():
                fire_gathers(h_in, slot_n, rbase_n)

            wait_gathers(h_in)

            for i in range(NDESC):
                for g in range(ROW_W // LANES):
                    w16 = wbuf[slot * NDESC + i, pl.ds(g * LANES, LANES)]
                    for l in range(LANES):
                        e = rbase + i * ROW_W + g * LANES + l
                        wsc = w16[l]
                        for kk in range(HALF // LANES):
                            sl = pl.ds(kk * LANES, LANES)
                            rows[e, sl] = rows[e, sl] * wsc
                pltpu.async_copy(rows.at[pl.ds(rbase + i * ROW_W, ROW_W)],
                                 acc.at[dstb.at[slot * NDESC + i]], ssem,
                                 add=True)

        wait_scatters()

    def zero_rows():
        @pl.loop(0, 2 * CHUNK_EDGES, unroll=4)
        def _(r):
            for k in range(HALF // LANES):
                rows[r, pl.ds(k * LANES, LANES)] = jnp.zeros((LANES,), jnp.float32)

    def zero_acc():
        nz = 2 * CHUNK_EDGES
        for j in range(ACC_ROWS_PER_TILE // nz):
            pltpu.sync_copy(rows, acc.at[pl.ds(wbase + j * nz, nz)])
        rem = ACC_ROWS_PER_TILE % nz
        pltpu.sync_copy(rows.at[pl.ds(0, rem)],
                        acc.at[pl.ds(wbase + ACC_ROWS_PER_TILE - rem, rem)])

    def writeout(h_out):
        pltpu.sync_copy(acc.at[pl.ds(wbase, ACC_ROWS_PER_TILE)],
                        h_out.at[pl.ds(coff + wbase, ACC_ROWS_PER_TILE)])

    def build_h0_part(b, sz):
        pltpu.sync_copy(emb_hbm.at[pl.ds(b, sz), pl.ds(c * HALF, HALF)],
                        rows.at[pl.ds(0, sz)])
        pltpu.sync_copy(rows.at[pl.ds(0, sz)], h0_hbm.at[pl.ds(coff + b, sz)])

    def build_h0():
        for j in range(4):
            build_h0_part(wbase + j * 768, 768)

        @pl.when(s < NS - 1)
        def _():
            build_h0_part(wbase + 3072, ACC_ROWS_PER_TILE - 3072)

        @pl.when(s == NS - 1)
        def _():
            build_h0_part(wbase + 3072, N_LAST - 3072)

    def mean_part(b, sz):
        third = jnp.float32(1.0 / 3.0)
        pltpu.sync_copy(emb_hbm.at[pl.ds(b, sz), pl.ds(c * HALF, HALF)],
                        rows.at[pl.ds(0, sz)])
        pltpu.sync_copy(h1_hbm.at[pl.ds(coff + b, sz)], rows.at[pl.ds(256, sz)])
        pltpu.sync_copy(acc.at[pl.ds(b, sz)], rows.at[pl.ds(512, sz)])

        @pl.loop(0, sz, unroll=4)
        def _(r):
            for k2 in range(HALF // LANES):
                sl = pl.ds(k2 * LANES, LANES)
                rows[r, sl] = (rows[r, sl] + rows[256 + r, sl]
                               + rows[512 + r, sl]) * third

        pltpu.sync_copy(rows.at[pl.ds(0, sz)],
                        out_hbm.at[pl.ds(b, sz), pl.ds(c * HALF, HALF)])

    def mean_out():
        @pl.loop(0, 12, unroll=1)
        def _(j):
            mean_part(wbase + j * 256, 256)

        @pl.when(s < NS - 1)
        def _():
            mean_part(wbase + 3072, ACC_ROWS_PER_TILE - 3072)

        @pl.when(s == NS - 1)
        def _():
            mean_part(wbase + 3072, N_LAST - 3072)

    build_h0()
    zero_rows()
    zero_acc()
    plsc.subcore_barrier()
    run_layer(h0_hbm)
    plsc.subcore_barrier()
    writeout(h1_hbm)
    zero_rows()
    zero_acc()
    plsc.subcore_barrier()
    run_layer(h1_hbm)
    plsc.subcore_barrier()
    mean_out()


_propagate = functools.partial(
    pl.kernel,
    out_type=(jax.ShapeDtypeStruct((2 * N_PAD, HALF), jnp.float32),
              jax.ShapeDtypeStruct((2 * N_PAD, HALF), jnp.float32),
              jax.ShapeDtypeStruct((N, D), jnp.float32)),
    mesh=plsc.VectorSubcoreMesh(core_axis_name="c", subcore_axis_name="s"),
    compiler_params=pltpu.CompilerParams(use_tc_tiling_on_sc=False),
    scratch_types=[
        pltpu.VMEM((ESLOTS * NDESC, ROW_W), jnp.int32),
        pltpu.VMEM((ESLOTS * NDESC, ROW_W), jnp.int32),
        pltpu.VMEM((ESLOTS * NDESC, ROW_W), jnp.float32),
        pltpu.VMEM((2 * CHUNK_EDGES, HALF), jnp.float32),
        pltpu.VMEM_SHARED((N_PAD, HALF), jnp.float32),
        pltpu.SemaphoreType.DMA,
        pltpu.SemaphoreType.DMA,
        pltpu.SemaphoreType.DMA,
    ],
)(_sc_body)


def kernel(edge_index, edge_weight, emb_weight):
    dst = edge_index[0].reshape(EROWS, ROW_W)
    src = edge_index[1].reshape(EROWS, ROW_W)
    w = edge_weight.reshape(EROWS, ROW_W)
    _h0, _h1, out = _propagate(dst, src, w, emb_weight)
    return out
